# Initial kernel scaffold; baseline (speedup 1.0000x reference)
import jax
import jax.numpy as jnp
from jax.experimental import pallas as pl


def kernel(data, table, W, b):
    raise NotImplementedError("write your pallas kernel here")


# speedup vs baseline: 21.1633x; 21.1633x over previous
import functools

import jax
import jax.numpy as jnp
from jax import lax
from jax.experimental import pallas as pl
from jax.experimental.pallas import tpu as pltpu
from jax.experimental.pallas import tpu_sc as plsc

VOCAB = 100000
EMB = 64
CLS = 4
BATCH = 4096
HIST = 200

LANES = 16
PACK = 8
KDIM = EMB * PACK
NDIM = LANES * PACK
MROWS = VOCAB // PACK

NWORKERS = 32
ITEMS = BATCH // NWORKERS
NSLOTS = 8
NGROUPS = HIST // NSLOTS


def _project_body(t_ref, w_ref, bias_ref, o_ref):
    o_ref[...] = (
        jnp.dot(t_ref[...], w_ref[...], preferred_element_type=jnp.float32)
        + bias_ref[...]
    )


def _project(table_r, w_blk, bias_row):
    bm = 1024
    grid = pl.cdiv(MROWS, bm)
    return pl.pallas_call(
        _project_body,
        grid=(grid,),
        in_specs=[
            pl.BlockSpec((bm, KDIM), lambda i: (i, 0)),
            pl.BlockSpec((KDIM, NDIM), lambda i: (0, 0)),
            pl.BlockSpec((1, NDIM), lambda i: (0, 0)),
        ],
        out_specs=pl.BlockSpec((bm, NDIM), lambda i: (i, 0)),
        out_shape=jax.ShapeDtypeStruct((MROWS, NDIM), jnp.float32),
    )(table_r, w_blk, bias_row)


def _sc_body(p16_hbm, idx_hbm, out_hbm, idx_v, accs, sems):
    wid = lax.axis_index("s") * 2 + lax.axis_index("c")
    base = wid * ITEMS

    pltpu.sync_copy(idx_hbm.at[wid], idx_v)

    for g in range(NSLOTS):
        pltpu.async_copy(p16_hbm.at[idx_v.at[g]], accs[g], sems[g])

    @pl.loop(1, NGROUPS)
    def _(i):
        for g in range(NSLOTS):
            l = i * NSLOTS + g
            pltpu.make_async_copy(p16_hbm.at[idx_v.at[g]], accs[g], sems[g]).wait()
            pltpu.async_copy(p16_hbm.at[idx_v.at[l]], accs[g], sems[g], add=True)

    for g in range(NSLOTS):
        pltpu.make_async_copy(p16_hbm.at[idx_v.at[g]], accs[g], sems[g]).wait()

    @pl.loop(0, ITEMS)
    def _(j):
        tot = accs[0][j]
        for g in range(1, NSLOTS):
            tot = tot + accs[g][j]
        accs[0][j] = tot

    pltpu.sync_copy(accs[0], out_hbm.at[pl.ds(base, ITEMS)])


def _pooled_lookup(p16, idx3):
    mesh = plsc.VectorSubcoreMesh(core_axis_name="c", subcore_axis_name="s")
    scratch = (
        pltpu.VMEM((HIST, ITEMS), jnp.int32),
        tuple(pltpu.VMEM((ITEMS, LANES), jnp.float32) for _ in range(NSLOTS)),
        tuple(pltpu.SemaphoreType.DMA for _ in range(NSLOTS)),
    )
    f = pl.kernel(
        _sc_body,
        out_type=jax.ShapeDtypeStruct((BATCH, LANES), jnp.float32),
        mesh=mesh,
        scratch_types=scratch,
        compiler_params=pltpu.CompilerParams(use_tc_tiling_on_sc=False),
    )
    return f(p16, idx3)


@jax.jit
def kernel(data, table, W, b):
    w_pad = jnp.zeros((EMB, LANES), jnp.float32).at[:, :CLS].set(W.T / HIST)
    w_blk = jnp.kron(jnp.eye(PACK, dtype=jnp.float32), w_pad)
    b_pad = jnp.zeros((LANES,), jnp.float32).at[:CLS].set(b / HIST)
    bias_row = jnp.tile(b_pad, PACK)[None]

    table_r = table.reshape(MROWS, KDIM)
    p16 = _project(table_r, w_blk, bias_row).reshape(VOCAB, LANES)

    idx3 = (
        data.T.reshape(HIST, NWORKERS, ITEMS).transpose(1, 0, 2)
    )

    out16 = _pooled_lookup(p16, idx3)
    return out16[:, :CLS]
